# baseline (speedup 1.0000x reference)
import functools

import jax
import jax.numpy as jnp
from jax import lax
from jax.experimental import pallas as pl
from jax.experimental.pallas import tpu as pltpu
from jax.experimental.pallas import tpu_sc as plsc

_BATCH = 16384
_D = 64
_NC = 2
_NS = 16
_NW = _NC * _NS
_BPW = _BATCH // _NW
_CHUNK = 128
_NCH = _BPW // _CHUNK


def _mf_body(uid_hbm, iid_hbm, uemb_hbm, iemb_hbm, ub_hbm, ib_hbm, gb_hbm,
             out_hbm,
             uid_v, iid_v, urows_v, irows_v, ub_v, ib_v, gb_v, out_v, sem):
  wid = lax.axis_index("s") * _NC + lax.axis_index("c")
  base = wid * _BPW

  for c in range(_NCH):
    pltpu.sync_copy(uid_hbm.at[pl.ds(base + c * _CHUNK, _CHUNK)], uid_v.at[c])
    pltpu.sync_copy(iid_hbm.at[pl.ds(base + c * _CHUNK, _CHUNK)], iid_v.at[c])
  pltpu.sync_copy(gb_hbm, gb_v)

  copies = []
  for c in range(_NCH):
    sl = pl.ds(c * _CHUNK, _CHUNK)
    copies.append(pltpu.async_copy(uemb_hbm.at[uid_v.at[c]], urows_v.at[sl], sem))
    copies.append(pltpu.async_copy(iemb_hbm.at[iid_v.at[c]], irows_v.at[sl], sem))
    copies.append(pltpu.async_copy(ub_hbm.at[uid_v.at[c]], ub_v.at[sl], sem))
    copies.append(pltpu.async_copy(ib_hbm.at[iid_v.at[c]], ib_v.at[sl], sem))
  for cp in copies:
    cp.wait()

  gbvec = gb_v[...]

  def group(g, carry):
    rbase = g * 16
    rows = rbase + lax.iota(jnp.int32, 16)
    acc = jnp.zeros((16,), jnp.float32)
    for j in range(_D):
      cols = jnp.full((16,), j, jnp.int32)
      u = plsc.load_gather(urows_v, [rows, cols])
      w = plsc.load_gather(irows_v, [rows, cols])
      acc = acc + u * w
    res = acc + ub_v[pl.ds(rbase, 16)] + ib_v[pl.ds(rbase, 16)] + gbvec
    out_v[pl.ds(rbase, 16)] = res
    return carry

  lax.fori_loop(0, _BPW // 16, group, 0)

  pltpu.sync_copy(out_v, out_hbm.at[pl.ds(base, _BPW)])


@functools.partial(jax.jit, static_argnames=())
def _mf(uid, iid, uemb, iemb, ub, ib, gb):
  mesh = plsc.VectorSubcoreMesh(core_axis_name="c", subcore_axis_name="s")
  f = functools.partial(
      pl.kernel,
      out_type=jax.ShapeDtypeStruct((_BATCH,), jnp.float32),
      mesh=mesh,
      compiler_params=pltpu.CompilerParams(
          needs_layout_passes=False, use_tc_tiling_on_sc=False),
      scratch_types=[
          pltpu.VMEM((_NCH, _CHUNK), jnp.int32),
          pltpu.VMEM((_NCH, _CHUNK), jnp.int32),
          pltpu.VMEM((_BPW, _D), jnp.float32),
          pltpu.VMEM((_BPW, _D), jnp.float32),
          pltpu.VMEM((_BPW,), jnp.float32),
          pltpu.VMEM((_BPW,), jnp.float32),
          pltpu.VMEM((16,), jnp.float32),
          pltpu.VMEM((_BPW,), jnp.float32),
          pltpu.SemaphoreType.DMA,
      ],
  )(_mf_body)
  return f(uid, iid, uemb, iemb, ub, ib, gb)


def kernel(user_ids, item_ids, user_embeddings, item_embeddings, user_bias,
           item_bias, global_bias):
  uid = user_ids.astype(jnp.int32)
  iid = item_ids.astype(jnp.int32)
  ub = user_bias.reshape(-1)
  ib = item_bias.reshape(-1)
  gb16 = jnp.broadcast_to(global_bias.reshape(-1), (16,))
  return _mf(uid, iid, user_embeddings, item_embeddings, ub, ib, gb16)
